# baseline (speedup 1.0000x reference)
import functools

import jax
import jax.numpy as jnp
from jax import lax
from jax.experimental import pallas as pl
from jax.experimental.pallas import tpu as pltpu
from jax.experimental.pallas import tpu_sc as plsc

ROWS = 128
COLS = 32768
LANES = 16
NUM_CORES = 2
NUM_SUBCORES = 16
NUM_WORKERS = NUM_CORES * NUM_SUBCORES
ROWS_PER_WORKER = ROWS // NUM_WORKERS
CHUNK = 4096
NUM_CHUNKS = COLS // CHUNK
VREGS_PER_CHUNK = CHUNK // LANES


def _sc_masked_cumsum(x_hbm, m_hbm, out_hbm,
                      xb0, xb1, mb0, mb1, sem_in0, sem_in1, sem_out):
    wid = lax.axis_index("s") * NUM_CORES + lax.axis_index("c")
    row0 = wid * ROWS_PER_WORKER
    xb = (xb0, xb1)
    mb = (mb0, mb1)
    sem_in = (sem_in0, sem_in1)

    def start_in(c, s):
        col = pl.ds(c * CHUNK, CHUNK)
        h = []
        for r in range(ROWS_PER_WORKER):
            h.append(pltpu.async_copy(x_hbm.at[row0 + r, col], xb[s].at[r],
                                      sem_in[s]))
            h.append(pltpu.async_copy(m_hbm.at[row0 + r, col], mb[s].at[r],
                                      sem_in[s]))
        return h

    def start_out(c, s):
        col = pl.ds(c * CHUNK, CHUNK)
        return [pltpu.async_copy(xb[s].at[r], out_hbm.at[row0 + r, col],
                                 sem_out)
                for r in range(ROWS_PER_WORKER)]

    carries = (jnp.zeros((LANES,), jnp.float32),) * ROWS_PER_WORKER
    in_h = {0: start_in(0, 0)}
    out_h = {}
    for c in range(NUM_CHUNKS):
        s = c & 1
        if c + 1 < NUM_CHUNKS:
            if c - 1 >= 0:
                for h in out_h.pop(c - 1):
                    h.wait()
            in_h[c + 1] = start_in(c + 1, 1 - s)
        for h in in_h.pop(c):
            h.wait()

        xbuf, mbuf = xb[s], mb[s]

        @plsc.parallel_loop(0, VREGS_PER_CHUNK, 1, unroll=4, carry=carries)
        def body(j, carry, xbuf=xbuf, mbuf=mbuf):
            base = j * LANES
            last = jnp.full((LANES,), LANES - 1, jnp.int32)
            out = []
            for r in range(ROWS_PER_WORKER):
                sl = (r, pl.ds(base, LANES))
                v = xbuf[sl] * mbuf[sl]
                sc = plsc.cumsum(v)
                ov = sc + carry[r]
                xbuf[sl] = ov
                out.append(jnp.take_along_axis(
                    ov, last, axis=0, mode="promise_in_bounds"))
            return tuple(out)

        carries = body
        out_h[c] = start_out(c, s)
    for c in (NUM_CHUNKS - 2, NUM_CHUNKS - 1):
        for h in out_h.pop(c, ()):
            h.wait()


@jax.jit
def _masked_cumsum(x, mask_f32):
    mesh = plsc.VectorSubcoreMesh(core_axis_name="c", subcore_axis_name="s")
    kern = functools.partial(
        pl.kernel,
        out_type=jax.ShapeDtypeStruct((ROWS, COLS), jnp.float32),
        mesh=mesh,
        scratch_types=[
            pltpu.VMEM((ROWS_PER_WORKER, CHUNK), jnp.float32),
            pltpu.VMEM((ROWS_PER_WORKER, CHUNK), jnp.float32),
            pltpu.VMEM((ROWS_PER_WORKER, CHUNK), jnp.float32),
            pltpu.VMEM((ROWS_PER_WORKER, CHUNK), jnp.float32),
            pltpu.SemaphoreType.DMA,
            pltpu.SemaphoreType.DMA,
            pltpu.SemaphoreType.DMA,
        ],
        compiler_params=pltpu.CompilerParams(needs_layout_passes=False),
    )(_sc_masked_cumsum)
    return kern(x, mask_f32)


def kernel(x, mask):
    return _masked_cumsum(x, mask.astype(jnp.float32))


# speedup vs baseline: 1.0367x; 1.0367x over previous
import functools

import jax
import jax.numpy as jnp
from jax import lax
from jax.experimental import pallas as pl
from jax.experimental.pallas import tpu as pltpu
from jax.experimental.pallas import tpu_sc as plsc

ROWS = 128
COLS = 32768
LANES = 16
NUM_CORES = 2
NUM_SUBCORES = 16
NUM_WORKERS = NUM_CORES * NUM_SUBCORES
ROWS_PER_WORKER = ROWS // NUM_WORKERS
CHUNK = 4096
NUM_CHUNKS = COLS // CHUNK
VREGS_PER_CHUNK = CHUNK // LANES


def _sc_masked_cumsum(x_hbm, m_hbm, out_hbm,
                      xb0, xb1, mb0, mb1, sem_in0, sem_in1, sem_out):
    wid = lax.axis_index("s") * NUM_CORES + lax.axis_index("c")
    row0 = wid * ROWS_PER_WORKER
    xb = (xb0, xb1)
    mb = (mb0, mb1)
    sem_in = (sem_in0, sem_in1)

    def start_in(c, s):
        col = pl.ds(c * CHUNK, CHUNK)
        h = []
        for r in range(ROWS_PER_WORKER):
            h.append(pltpu.async_copy(x_hbm.at[row0 + r, col], xb[s].at[r],
                                      sem_in[s]))
            h.append(pltpu.async_copy(m_hbm.at[row0 + r, col], mb[s].at[r],
                                      sem_in[s]))
        return h

    def start_out(c, s):
        col = pl.ds(c * CHUNK, CHUNK)
        return [pltpu.async_copy(xb[s].at[r], out_hbm.at[row0 + r, col],
                                 sem_out)
                for r in range(ROWS_PER_WORKER)]

    carries = (jnp.zeros((LANES,), jnp.float32),) * ROWS_PER_WORKER
    in_h = {0: start_in(0, 0)}
    out_h = {}
    for c in range(NUM_CHUNKS):
        s = c & 1
        if c + 1 < NUM_CHUNKS:
            if c - 1 >= 0:
                for h in out_h.pop(c - 1):
                    h.wait()
            in_h[c + 1] = start_in(c + 1, 1 - s)
        for h in in_h.pop(c):
            h.wait()

        xbuf, mbuf = xb[s], mb[s]

        @plsc.parallel_loop(0, VREGS_PER_CHUNK, 1, unroll=2, carry=carries)
        def body(j, carry, xbuf=xbuf, mbuf=mbuf):
            base = j * LANES
            last = jnp.full((LANES,), LANES - 1, jnp.int32)
            out = []
            for r in range(ROWS_PER_WORKER):
                sl = (r, pl.ds(base, LANES))
                v = xbuf[sl] * mbuf[sl]
                sc = plsc.cumsum(v)
                ov = sc + carry[r]
                xbuf[sl] = ov
                out.append(jnp.take_along_axis(
                    ov, last, axis=0, mode="promise_in_bounds"))
            return tuple(out)

        carries = body
        out_h[c] = start_out(c, s)
    for c in (NUM_CHUNKS - 2, NUM_CHUNKS - 1):
        for h in out_h.pop(c, ()):
            h.wait()


@jax.jit
def _masked_cumsum(x, mask_f32):
    mesh = plsc.VectorSubcoreMesh(core_axis_name="c", subcore_axis_name="s")
    kern = functools.partial(
        pl.kernel,
        out_type=jax.ShapeDtypeStruct((ROWS, COLS), jnp.float32),
        mesh=mesh,
        scratch_types=[
            pltpu.VMEM((ROWS_PER_WORKER, CHUNK), jnp.float32),
            pltpu.VMEM((ROWS_PER_WORKER, CHUNK), jnp.float32),
            pltpu.VMEM((ROWS_PER_WORKER, CHUNK), jnp.float32),
            pltpu.VMEM((ROWS_PER_WORKER, CHUNK), jnp.float32),
            pltpu.SemaphoreType.DMA,
            pltpu.SemaphoreType.DMA,
            pltpu.SemaphoreType.DMA,
        ],
        compiler_params=pltpu.CompilerParams(needs_layout_passes=False),
    )(_sc_masked_cumsum)
    return kern(x, mask_f32)


def kernel(x, mask):
    return _masked_cumsum(x, mask.astype(jnp.float32))
